# baseline (speedup 1.0000x reference)
import functools

import jax
import jax.numpy as jnp
from jax.experimental import pallas as pl
from jax.experimental.pallas import tpu as pltpu

_K = 64


def _enc_body(x_ref, we_ref, be_ref, bd_ref, pre_ref, xs_ref):
    j = pl.program_id(0)

    @pl.when(j == 0)
    def _():
        xs_ref[:] = x_ref[:] - bd_ref[:]

    acc = jnp.dot(xs_ref[:], we_ref[:], preferred_element_type=jnp.float32)
    pre_ref[:] = jnp.maximum(acc + be_ref[:], 0.0)


def _mask_body(pre_ref, z_ref, *, k, n_c):
    z = pre_ref[:]
    bits = jax.lax.bitcast_convert_type(z, jnp.int32)
    n = z.shape[1]
    r = z.shape[0]
    blk_c = n // n_c

    ones_col = jnp.ones((n, 8), jnp.float32)

    def step(it, lo):
        cand = lo | (jnp.int32(1) << (30 - it))
        m = jnp.where(bits >= cand, 1.0, 0.0)
        cnt = jnp.dot(m, ones_col,
                      preferred_element_type=jnp.float32)[:, :1]
        return jnp.where(cnt >= k, cand, lo)

    lo = jax.lax.fori_loop(0, 31, step, jnp.zeros((r, 1), jnp.int32))
    z_ref[:] = jnp.where(bits >= lo, z, 0.0).astype(jnp.bfloat16)


def _dec_body(z_ref, wd_ref, bd_ref, o_ref):
    kk = pl.program_id(0)

    @pl.when(kk == 0)
    def _():
        o_ref[:] = jnp.zeros_like(o_ref) + bd_ref[:]

    o_ref[:] += jnp.dot(z_ref[:], wd_ref[:], preferred_element_type=jnp.float32)


def kernel(x, W_enc, b_enc, W_dec, b_dec):
    b, s, d_model = x.shape
    d_sae = W_enc.shape[1]
    rows = b * s
    x_flat = x.reshape(rows, d_model)

    blk_j = min(512, d_sae)
    n_j = d_sae // blk_j

    pre = pl.pallas_call(
        _enc_body,
        grid=(n_j,),
        in_specs=[
            pl.BlockSpec((rows, d_model), lambda j: (0, 0)),
            pl.BlockSpec((d_model, blk_j), lambda j: (0, j)),
            pl.BlockSpec((1, blk_j), lambda j: (0, j)),
            pl.BlockSpec((1, d_model), lambda j: (0, 0)),
        ],
        out_specs=pl.BlockSpec((rows, blk_j), lambda j: (0, j)),
        out_shape=jax.ShapeDtypeStruct((rows, d_sae), jnp.float32),
        scratch_shapes=[pltpu.VMEM((rows, d_model), jnp.float32)],
        compiler_params=pltpu.CompilerParams(
            dimension_semantics=("arbitrary",)),
    )(x_flat, W_enc, b_enc.reshape(1, d_sae), b_dec.reshape(1, d_model))

    blk_i = min(128, rows)
    n_i = rows // blk_i
    z = pl.pallas_call(
        functools.partial(_mask_body, k=_K, n_c=4),
        grid=(n_i,),
        in_specs=[pl.BlockSpec((blk_i, d_sae), lambda i: (i, 0))],
        out_specs=pl.BlockSpec((blk_i, d_sae), lambda i: (i, 0)),
        out_shape=jax.ShapeDtypeStruct((rows, d_sae), jnp.bfloat16),
        compiler_params=pltpu.CompilerParams(
            dimension_semantics=("arbitrary",)),
    )(pre)

    wd16 = W_dec.astype(jnp.bfloat16)
    blk_k = min(512, d_sae)
    n_k = d_sae // blk_k
    x_hat = pl.pallas_call(
        _dec_body,
        grid=(n_k,),
        in_specs=[
            pl.BlockSpec((rows, blk_k), lambda kk: (0, kk)),
            pl.BlockSpec((blk_k, d_model), lambda kk: (kk, 0)),
            pl.BlockSpec((1, d_model), lambda kk: (0, 0)),
        ],
        out_specs=pl.BlockSpec((rows, d_model), lambda kk: (0, 0)),
        out_shape=jax.ShapeDtypeStruct((rows, d_model), jnp.float32),
        compiler_params=pltpu.CompilerParams(
            dimension_semantics=("arbitrary",)),
    )(z, wd16, b_dec.reshape(1, d_model))

    return x_hat.reshape(b, s, d_model)


# speedup vs baseline: 1.1964x; 1.1964x over previous
import functools

import jax
import jax.numpy as jnp
from jax.experimental import pallas as pl
from jax.experimental.pallas import tpu as pltpu

_K = 64


def _enc_body(x_ref, we_ref, be_ref, bd_ref, pre_ref, xs_ref):
    j = pl.program_id(0)

    @pl.when(j == 0)
    def _():
        xs_ref[:] = x_ref[:] - bd_ref[:]

    acc = jnp.dot(xs_ref[:], we_ref[:], preferred_element_type=jnp.float32)
    pre_ref[:] = jnp.maximum(acc + be_ref[:], 0.0)


def _mask_body(pre_ref, z_ref, *, k, n_c):
    z = pre_ref[:]
    bits = jax.lax.bitcast_convert_type(z, jnp.int32)
    n = z.shape[1]
    r = z.shape[0]
    blk_c = n // n_c

    def step(it, lo):
        cand = lo | (jnp.int32(1) << (30 - it))
        cnt = jnp.zeros((r, 1), jnp.int32)
        for c in range(n_c):
            cnt += jnp.sum(
                (bits[:, c * blk_c:(c + 1) * blk_c] >= cand).astype(jnp.int32),
                axis=1, keepdims=True)
        return jnp.where(cnt >= k, cand, lo)

    lo = jax.lax.fori_loop(0, 31, step, jnp.zeros((r, 1), jnp.int32))
    z_ref[:] = jnp.where(bits >= lo, z, 0.0).astype(jnp.bfloat16)


def _dec_body(z_ref, wd_ref, bd_ref, o_ref):
    kk = pl.program_id(0)

    @pl.when(kk == 0)
    def _():
        o_ref[:] = jnp.zeros_like(o_ref) + bd_ref[:]

    o_ref[:] += jnp.dot(z_ref[:], wd_ref[:], preferred_element_type=jnp.float32)


def kernel(x, W_enc, b_enc, W_dec, b_dec):
    b, s, d_model = x.shape
    d_sae = W_enc.shape[1]
    rows = b * s
    x_flat = x.reshape(rows, d_model)

    blk_j = min(512, d_sae)
    n_j = d_sae // blk_j

    pre = pl.pallas_call(
        _enc_body,
        grid=(n_j,),
        in_specs=[
            pl.BlockSpec((rows, d_model), lambda j: (0, 0)),
            pl.BlockSpec((d_model, blk_j), lambda j: (0, j)),
            pl.BlockSpec((1, blk_j), lambda j: (0, j)),
            pl.BlockSpec((1, d_model), lambda j: (0, 0)),
        ],
        out_specs=pl.BlockSpec((rows, blk_j), lambda j: (0, j)),
        out_shape=jax.ShapeDtypeStruct((rows, d_sae), jnp.float32),
        scratch_shapes=[pltpu.VMEM((rows, d_model), jnp.float32)],
        compiler_params=pltpu.CompilerParams(
            dimension_semantics=("arbitrary",)),
    )(x_flat, W_enc, b_enc.reshape(1, d_sae), b_dec.reshape(1, d_model))

    blk_i = min(128, rows)
    n_i = rows // blk_i
    z = pl.pallas_call(
        functools.partial(_mask_body, k=_K, n_c=4),
        grid=(n_i,),
        in_specs=[pl.BlockSpec((blk_i, d_sae), lambda i: (i, 0))],
        out_specs=pl.BlockSpec((blk_i, d_sae), lambda i: (i, 0)),
        out_shape=jax.ShapeDtypeStruct((rows, d_sae), jnp.bfloat16),
        compiler_params=pltpu.CompilerParams(
            dimension_semantics=("arbitrary",)),
    )(pre)

    wd16 = W_dec.astype(jnp.bfloat16)
    blk_k = min(512, d_sae)
    n_k = d_sae // blk_k
    x_hat = pl.pallas_call(
        _dec_body,
        grid=(n_k,),
        in_specs=[
            pl.BlockSpec((rows, blk_k), lambda kk: (0, kk)),
            pl.BlockSpec((blk_k, d_model), lambda kk: (kk, 0)),
            pl.BlockSpec((1, d_model), lambda kk: (0, 0)),
        ],
        out_specs=pl.BlockSpec((rows, d_model), lambda kk: (0, 0)),
        out_shape=jax.ShapeDtypeStruct((rows, d_model), jnp.float32),
        compiler_params=pltpu.CompilerParams(
            dimension_semantics=("arbitrary",)),
    )(z, wd16, b_dec.reshape(1, d_model))

    return x_hat.reshape(b, s, d_model)
